# Initial kernel scaffold; baseline (speedup 1.0000x reference)
import jax
import jax.numpy as jnp
from jax.experimental import pallas as pl


def kernel(edge_attr, table):
    raise NotImplementedError("write your pallas kernel here")


# speedup vs baseline: 4.0804x; 4.0804x over previous
import functools

import jax
import jax.numpy as jnp
from jax import lax
from jax.experimental import pallas as pl
from jax.experimental.pallas import tpu as pltpu
from jax.experimental.pallas import tpu_sc as plsc

N_EDGES = 3_200_000
NUM_TYPES = 64
EMB = 16

NC = 2
NS = 16
NW = NC * NS
B_PER_W = N_EDGES // NW
CHUNK = 2000
N_CHUNKS = B_PER_W // CHUNK

_mesh = plsc.VectorSubcoreMesh(core_axis_name="c", subcore_axis_name="s")


@functools.partial(
    pl.kernel,
    mesh=_mesh,
    out_type=jax.ShapeDtypeStruct((N_EDGES, EMB), jnp.float32),
    scratch_types=[
        pltpu.VMEM((CHUNK,), jnp.int32),
        pltpu.VMEM((CHUNK, EMB), jnp.float32),
        pltpu.SemaphoreType.DMA,
    ],
    compiler_params=pltpu.CompilerParams(use_tc_tiling_on_sc=False),
)
def _gather_kernel(idx_hbm, table_hbm, out_hbm, idx_v, rows_v, sem):
    wid = lax.axis_index("s") * NC + lax.axis_index("c")
    base0 = wid * B_PER_W

    def body(i, carry):
        base = base0 + i * CHUNK
        pltpu.sync_copy(idx_hbm.at[pl.ds(base, CHUNK)], idx_v)
        pltpu.async_copy(table_hbm.at[idx_v], rows_v, sem).wait()
        pltpu.sync_copy(rows_v, out_hbm.at[pl.ds(base, CHUNK)])
        return carry

    lax.fori_loop(0, N_CHUNKS, body, 0)


def kernel(edge_attr, table):
    return _gather_kernel(edge_attr, table)


# speedup vs baseline: 5.7797x; 1.4164x over previous
import functools

import jax
import jax.numpy as jnp
from jax import lax
from jax.experimental import pallas as pl
from jax.experimental.pallas import tpu as pltpu
from jax.experimental.pallas import tpu_sc as plsc

N_EDGES = 3_200_000
NUM_TYPES = 64
EMB = 16
LANES = 16

NC = 2
NS = 16
NW = NC * NS
B_PER_W = N_EDGES // NW
CHUNK = 2000
N_CHUNKS = B_PER_W // CHUNK
GROUPS = CHUNK // LANES

_mesh = plsc.VectorSubcoreMesh(core_axis_name="c", subcore_axis_name="s")


@functools.partial(
    pl.kernel,
    mesh=_mesh,
    out_type=jax.ShapeDtypeStruct((N_EDGES * EMB,), jnp.float32),
    scratch_types=[
        pltpu.VMEM((NUM_TYPES * EMB,), jnp.float32),
        pltpu.VMEM((CHUNK,), jnp.int32),
        pltpu.VMEM((CHUNK * EMB,), jnp.float32),
    ],
    compiler_params=pltpu.CompilerParams(
        use_tc_tiling_on_sc=False, needs_layout_passes=False
    ),
)
def _emb_kernel(idx_hbm, table_hbm, out_hbm, table_v, idx_v, out_v):
    wid = lax.axis_index("s") * NC + lax.axis_index("c")
    base0 = wid * B_PER_W
    pltpu.sync_copy(table_hbm, table_v)
    jlane = lax.iota(jnp.int32, LANES) * EMB

    def chunk_body(i, carry):
        base = base0 + i * CHUNK
        pltpu.sync_copy(idx_hbm.at[pl.ds(base, CHUNK)], idx_v)

        def group_body(g, carry2):
            idx16 = idx_v[pl.ds(g * LANES, LANES)]
            flat = idx16 * EMB
            obase = jlane + g * (LANES * EMB)
            for c in range(EMB):
                vals = plsc.load_gather(table_v, [flat + c])
                plsc.store_scatter(out_v, [obase + c], vals)
            return carry2

        lax.fori_loop(0, GROUPS, group_body, 0)
        pltpu.sync_copy(out_v, out_hbm.at[pl.ds(base * EMB, CHUNK * EMB)])
        return carry

    lax.fori_loop(0, N_CHUNKS, chunk_body, 0)


def kernel(edge_attr, table):
    out = _emb_kernel(edge_attr, table.reshape(-1))
    return out.reshape(N_EDGES, EMB)


# speedup vs baseline: 13.5589x; 2.3459x over previous
import functools

import jax
import jax.numpy as jnp
from jax import lax
from jax.experimental import pallas as pl
from jax.experimental.pallas import tpu as pltpu
from jax.experimental.pallas import tpu_sc as plsc

N_EDGES = 3_200_000
NUM_TYPES = 64
EMB = 16
LANES = 16

NC = 2
NS = 16
NW = NC * NS
CHUNK = 1024
N_CHUNKS = N_EDGES // CHUNK
CHUNKS_PER_W = -(-N_CHUNKS // NW)
GROUPS = CHUNK // LANES

_mesh = plsc.VectorSubcoreMesh(core_axis_name="c", subcore_axis_name="s")


@functools.partial(
    pl.kernel,
    mesh=_mesh,
    out_type=jax.ShapeDtypeStruct((EMB, N_EDGES), jnp.float32),
    scratch_types=[
        pltpu.VMEM((NUM_TYPES * EMB,), jnp.float32),
        pltpu.VMEM((CHUNK,), jnp.int32),
        pltpu.VMEM((EMB, CHUNK), jnp.float32),
    ],
    compiler_params=pltpu.CompilerParams(needs_layout_passes=False),
)
def _emb_kernel(idx_hbm, table_hbm, out_hbm, table_v, idx_v, out_v):
    wid = lax.axis_index("s") * NC + lax.axis_index("c")
    pltpu.sync_copy(table_hbm, table_v)

    def chunk_body(i, carry):
        cid = wid + i * NW

        @pl.when(cid < N_CHUNKS)
        def _():
            base = cid * CHUNK
            pltpu.sync_copy(idx_hbm.at[pl.ds(base, CHUNK)], idx_v)

            def group_body(g, carry2):
                idx16 = idx_v[pl.ds(g * LANES, LANES)]
                flat = idx16 * EMB
                for c in range(EMB):
                    vals = plsc.load_gather(table_v, [flat + c])
                    out_v[c, pl.ds(g * LANES, LANES)] = vals
                return carry2

            lax.fori_loop(0, GROUPS, group_body, 0)
            pltpu.sync_copy(out_v, out_hbm.at[:, pl.ds(base, CHUNK)])

        return carry

    lax.fori_loop(0, CHUNKS_PER_W, chunk_body, 0)


def kernel(edge_attr, table):
    return _emb_kernel(edge_attr, table.reshape(-1)).T


# speedup vs baseline: 19.5812x; 1.4442x over previous
import functools

import jax
import jax.numpy as jnp
from jax import lax
from jax.experimental import pallas as pl
from jax.experimental.pallas import tpu as pltpu
from jax.experimental.pallas import tpu_sc as plsc

N_EDGES = 3_200_000
NUM_TYPES = 64
EMB = 16
LANES = 16

NC = 2
NS = 16
NW = NC * NS
CHUNK = 1024
N_CHUNKS = N_EDGES // CHUNK
BASE_CHUNKS = N_CHUNKS // NW
EXTRA_W = N_CHUNKS - BASE_CHUNKS * NW
GROUPS = CHUNK // LANES
PAIRS = (BASE_CHUNKS - 1) // 2

_mesh = plsc.VectorSubcoreMesh(core_axis_name="c", subcore_axis_name="s")


@functools.partial(
    pl.kernel,
    mesh=_mesh,
    out_type=jax.ShapeDtypeStruct((EMB, N_EDGES), jnp.float32),
    scratch_types=[
        pltpu.VMEM((NUM_TYPES * EMB,), jnp.float32),
        pltpu.VMEM((CHUNK,), jnp.int32),
        pltpu.VMEM((CHUNK,), jnp.int32),
        pltpu.VMEM((EMB, CHUNK), jnp.float32),
        pltpu.VMEM((EMB, CHUNK), jnp.float32),
        pltpu.SemaphoreType.DMA,
        pltpu.SemaphoreType.DMA,
        pltpu.SemaphoreType.DMA,
        pltpu.SemaphoreType.DMA,
    ],
    compiler_params=pltpu.CompilerParams(needs_layout_passes=False),
)
def _emb_kernel(
    idx_hbm, table_hbm, out_hbm,
    table_v, idx_v0, idx_v1, out_v0, out_v1, isem0, isem1, osem0, osem1,
):
    wid = lax.axis_index("s") * NC + lax.axis_index("c")
    start = wid * BASE_CHUNKS + jnp.minimum(wid, EXTRA_W)
    extra = wid < EXTRA_W

    idx_bufs = (idx_v0, idx_v1)
    out_bufs = (out_v0, out_v1)
    isems = (isem0, isem1)
    osems = (osem0, osem1)

    pltpu.sync_copy(table_hbm, table_v)

    def idx_start(i, b):
        base = (start + i) * CHUNK
        pltpu.async_copy(idx_hbm.at[pl.ds(base, CHUNK)], idx_bufs[b], isems[b])

    def idx_wait(b):
        pltpu.make_async_copy(
            idx_hbm.at[pl.ds(0, CHUNK)], idx_bufs[b], isems[b]
        ).wait()

    def out_start(i, b):
        base = (start + i) * CHUNK
        pltpu.async_copy(
            out_bufs[b], out_hbm.at[:, pl.ds(base, CHUNK)], osems[b]
        )

    def out_wait(b):
        pltpu.make_async_copy(
            out_bufs[b], out_hbm.at[:, pl.ds(0, CHUNK)], osems[b]
        ).wait()

    def compute(b):
        idx_v = idx_bufs[b]
        out_v = out_bufs[b]

        def group_body(g, carry2):
            o = g * (2 * LANES)
            idx_a = idx_v[pl.ds(o, LANES)]
            idx_b = idx_v[pl.ds(o + LANES, LANES)]
            flat_a = idx_a * EMB
            flat_b = idx_b * EMB
            for c in range(EMB):
                va = plsc.load_gather(table_v, [flat_a + c])
                vb = plsc.load_gather(table_v, [flat_b + c])
                out_v[c, pl.ds(o, LANES)] = va
                out_v[c, pl.ds(o + LANES, LANES)] = vb
            return carry2

        lax.fori_loop(0, GROUPS // 2, group_body, 0)

    def step(i, b):
        @pl.when((i + 1 < BASE_CHUNKS) | extra)
        def _():
            idx_start(i + 1, b ^ 1)

        idx_wait(b)

        @pl.when(i >= 2)
        def _():
            out_wait(b)

        compute(b)
        out_start(i, b)

    idx_start(0, 0)

    def pair_body(k, carry):
        step(2 * k, 0)
        step(2 * k + 1, 1)
        return carry

    lax.fori_loop(0, PAIRS, pair_body, 0)

    last = BASE_CHUNKS - 1

    @pl.when(extra)
    def _():
        idx_start(last + 1, 1)

    idx_wait(0)
    out_wait(0)
    compute(0)
    out_start(last, 0)

    out_wait(1)

    @pl.when(extra)
    def _():
        idx_wait(1)
        compute(1)
        out_start(last + 1, 1)

    out_wait(0)

    @pl.when(extra)
    def _():
        out_wait(1)


def kernel(edge_attr, table):
    return _emb_kernel(edge_attr, table.reshape(-1)).T


# speedup vs baseline: 48.0457x; 2.4537x over previous
import functools

import jax
import jax.numpy as jnp
from jax import lax
from jax.experimental import pallas as pl
from jax.experimental.pallas import tpu as pltpu
from jax.experimental.pallas import tpu_sc as plsc

N_EDGES = 3_200_000
NUM_TYPES = 64
EMB = 16
LANES = 16

NC = 2
NS = 16
NW = NC * NS
CHUNK = 1024
N_CHUNKS = N_EDGES // CHUNK
BASE_CHUNKS = N_CHUNKS // NW
EXTRA_W = N_CHUNKS - BASE_CHUNKS * NW
GROUPS = CHUNK // LANES
PAIRS = (BASE_CHUNKS - 1) // 2

_mesh = plsc.VectorSubcoreMesh(core_axis_name="c", subcore_axis_name="s")


@functools.partial(
    pl.kernel,
    mesh=_mesh,
    out_type=jax.ShapeDtypeStruct((EMB, N_EDGES), jnp.float32),
    scratch_types=[
        pltpu.VMEM((NUM_TYPES * EMB,), jnp.float32),
        pltpu.VMEM((CHUNK,), jnp.int32),
        pltpu.VMEM((CHUNK,), jnp.int32),
        pltpu.VMEM((EMB, CHUNK), jnp.float32),
        pltpu.VMEM((EMB, CHUNK), jnp.float32),
        pltpu.SemaphoreType.DMA,
        pltpu.SemaphoreType.DMA,
        pltpu.SemaphoreType.DMA,
        pltpu.SemaphoreType.DMA,
    ],
    compiler_params=pltpu.CompilerParams(needs_layout_passes=False),
)
def _emb_kernel(
    idx_hbm, table_hbm, out_hbm,
    table_v, idx_v0, idx_v1, out_v0, out_v1, isem0, isem1, osem0, osem1,
):
    wid = lax.axis_index("s") * NC + lax.axis_index("c")
    start = wid * BASE_CHUNKS + jnp.minimum(wid, EXTRA_W)
    extra = wid < EXTRA_W

    idx_bufs = (idx_v0, idx_v1)
    out_bufs = (out_v0, out_v1)
    isems = (isem0, isem1)
    osems = (osem0, osem1)

    pltpu.sync_copy(table_hbm, table_v)

    def idx_start(i, b):
        base = (start + i) * CHUNK
        pltpu.async_copy(idx_hbm.at[pl.ds(base, CHUNK)], idx_bufs[b], isems[b])

    def idx_wait(b):
        pltpu.make_async_copy(
            idx_hbm.at[pl.ds(0, CHUNK)], idx_bufs[b], isems[b]
        ).wait()

    def out_start(i, b):
        base = (start + i) * CHUNK
        pltpu.async_copy(
            out_bufs[b], out_hbm.at[:, pl.ds(base, CHUNK)], osems[b]
        )

    def out_wait(b):
        pltpu.make_async_copy(
            out_bufs[b], out_hbm.at[:, pl.ds(0, CHUNK)], osems[b]
        ).wait()

    def compute(b):
        idx_v = idx_bufs[b]
        out_v = out_bufs[b]

        def group_body(g, carry2):
            o = g * (2 * LANES)
            flat_a = idx_v[pl.ds(o, LANES)]
            flat_b = idx_v[pl.ds(o + LANES, LANES)]
            for c in range(EMB):
                va = plsc.load_gather(table_v, [flat_a + c * NUM_TYPES])
                vb = plsc.load_gather(table_v, [flat_b + c * NUM_TYPES])
                out_v[c, pl.ds(o, LANES)] = va
                out_v[c, pl.ds(o + LANES, LANES)] = vb
            return carry2

        lax.fori_loop(0, GROUPS // 2, group_body, 0)

    def step(i, b):
        @pl.when((i + 1 < BASE_CHUNKS) | extra)
        def _():
            idx_start(i + 1, b ^ 1)

        idx_wait(b)

        @pl.when(i >= 2)
        def _():
            out_wait(b)

        compute(b)
        out_start(i, b)

    idx_start(0, 0)

    def pair_body(k, carry):
        step(2 * k, 0)
        step(2 * k + 1, 1)
        return carry

    lax.fori_loop(0, PAIRS, pair_body, 0)

    last = BASE_CHUNKS - 1

    @pl.when(extra)
    def _():
        idx_start(last + 1, 1)

    idx_wait(0)
    out_wait(0)
    compute(0)
    out_start(last, 0)

    out_wait(1)

    @pl.when(extra)
    def _():
        idx_wait(1)
        compute(1)
        out_start(last + 1, 1)

    out_wait(0)

    @pl.when(extra)
    def _():
        out_wait(1)


def kernel(edge_attr, table):
    return _emb_kernel(edge_attr, table.T.reshape(-1)).T
